# Initial kernel scaffold; baseline (speedup 1.0000x reference)
import jax
import jax.numpy as jnp
from jax.experimental import pallas as pl


def kernel(x, embed, attn, weight, ln_scale, ln_bias, G_values, G_indices):
    raise NotImplementedError("write your pallas kernel here")


# speedup vs baseline: 2.0698x; 2.0698x over previous
import functools

import jax
import jax.numpy as jnp
from jax import lax
from jax.experimental import pallas as pl
from jax.experimental.pallas import tpu as pltpu
from jax.experimental.pallas import tpu_sc as plsc

N = 8192
E = 131072
OUT = 256
B = 16384

NC = 2
NS = 16
L = 16


_BM = 512
_BK = 1024
_NM = N // _BM
_NK = N // _BK


def _mm_l2_body(embed_blk, attn_blk, w_blk, h_out, l2_out, acc, l2_acc):
    m = pl.program_id(0)
    k = pl.program_id(1)

    @pl.when(k == 0)
    def _():
        acc[...] = jnp.zeros_like(acc)

    acc[...] += jnp.dot(embed_blk[...], w_blk[...],
                        preferred_element_type=jnp.float32)

    @pl.when(jnp.logical_and(m == 0, k == 0))
    def _():
        l2_acc[0, 0] = 0.0

    a = attn_blk[...]
    l2_acc[0, 0] += jnp.sum(a * a)

    @pl.when(k == _NK - 1)
    def _():
        h_out[...] = acc[...]

    @pl.when(jnp.logical_and(m == _NM - 1, k == _NK - 1))
    def _():
        l2_out[...] = jnp.full((1, 1), jnp.sqrt(l2_acc[0, 0]) * 0.001,
                               jnp.float32)


def _mm_l2(embed, attn, weight):
    return pl.pallas_call(
        _mm_l2_body,
        grid=(_NM, _NK),
        in_specs=[
            pl.BlockSpec((_BM, _BK), lambda m, k: (m, k)),
            pl.BlockSpec((_BM, _BK), lambda m, k: (m, k)),
            pl.BlockSpec((_BK, OUT), lambda m, k: (k, 0)),
        ],
        out_specs=[
            pl.BlockSpec((_BM, OUT), lambda m, k: (m, 0)),
            pl.BlockSpec((1, 1), lambda m, k: (0, 0)),
        ],
        out_shape=[
            jax.ShapeDtypeStruct((N, OUT), jnp.float32),
            jax.ShapeDtypeStruct((1, 1), jnp.float32),
        ],
        scratch_shapes=[
            pltpu.VMEM((_BM, OUT), jnp.float32),
            pltpu.SMEM((1, 1), jnp.float32),
        ],
    )(embed, attn, weight)


_LNB = 512


def _ln_body(agg0_blk, agg1_blk, scale_blk, bias_blk, out_blk):
    h = jnp.maximum(agg0_blk[...] + agg1_blk[...], 0.0)
    mean = jnp.mean(h, axis=-1, keepdims=True)
    cent = h - mean
    var = jnp.mean(cent * cent, axis=-1, keepdims=True)
    out_blk[...] = cent * lax.rsqrt(var + 1e-5) * scale_blk[...] + bias_blk[...]


def _layernorm(agg2, ln_scale, ln_bias):
    nb = N // _LNB
    return pl.pallas_call(
        _ln_body,
        grid=(nb,),
        in_specs=[
            pl.BlockSpec((_LNB, OUT), lambda i: (i, 0)),
            pl.BlockSpec((_LNB, OUT), lambda i: (i + nb, 0)),
            pl.BlockSpec((1, OUT), lambda i: (0, 0)),
            pl.BlockSpec((1, OUT), lambda i: (0, 0)),
        ],
        out_specs=pl.BlockSpec((_LNB, OUT), lambda i: (i, 0)),
        out_shape=jax.ShapeDtypeStruct((N, OUT), jnp.float32),
    )(agg2, agg2, ln_scale.reshape(1, OUT), ln_bias.reshape(1, OUT))


_EW_CH = 128
_EW_PER_W = E // (NC * NS)
_EW_NCH = _EW_PER_W // _EW_CH


def _edge_w_kernel(row_hbm, col_hbm, gv_hbm, attn_hbm, out_hbm,
                   row_v, col_v, fidx_v, av_v, ew_v, sem):
    wid = lax.axis_index("s") * NC + lax.axis_index("c")
    base0 = wid * _EW_PER_W

    def chunk(j, carry):
        base = base0 + j * _EW_CH
        pltpu.sync_copy(row_hbm.at[pl.ds(base, _EW_CH)], row_v)
        pltpu.sync_copy(col_hbm.at[pl.ds(base, _EW_CH)], col_v)
        pltpu.sync_copy(gv_hbm.at[pl.ds(base, _EW_CH)], ew_v)

        def mk_idx(i, c2):
            sl = pl.ds(i * L, L)
            fidx_v[sl] = row_v[sl] * N + col_v[sl]
            return c2
        lax.fori_loop(0, _EW_CH // L, mk_idx, 0)

        pltpu.async_copy(attn_hbm.at[fidx_v], av_v, sem).wait()

        def mul(i, c2):
            sl = pl.ds(i * L, L)
            ew_v[sl] = ew_v[sl] * av_v[sl]
            return c2
        lax.fori_loop(0, _EW_CH // L, mul, 0)

        pltpu.sync_copy(ew_v, out_hbm.at[pl.ds(base, _EW_CH)])
        return carry

    lax.fori_loop(0, _EW_NCH, chunk, 0)


def _edge_weights(row, col, g_values, attn_flat):
    k = functools.partial(
        pl.kernel,
        out_type=jax.ShapeDtypeStruct((E,), jnp.float32),
        mesh=plsc.VectorSubcoreMesh(core_axis_name="c", subcore_axis_name="s"),
        scratch_types=[
            pltpu.VMEM((_EW_CH,), jnp.int32),
            pltpu.VMEM((_EW_CH,), jnp.int32),
            pltpu.VMEM((_EW_CH,), jnp.int32),
            pltpu.VMEM((_EW_CH,), jnp.float32),
            pltpu.VMEM((_EW_CH,), jnp.float32),
            pltpu.SemaphoreType.DMA,
        ],
    )(_edge_w_kernel)
    return k(row, col, g_values, attn_flat)


_SEG_CH = 128
_SEG_PER_T = E // (NC * NS)
_SEG_NCH = _SEG_PER_T // _SEG_CH
_ZCH = 128


def _seg_kernel(h_hbm, row_hbm, col_hbm, ew_hbm, agg_hbm,
                row_v, col_v, ew_v, sidx_v, rows_v, sem):
    c = lax.axis_index("c")
    s = lax.axis_index("s")

    def zrow(i, carry):
        def zlane(q, c2):
            rows_v[i, pl.ds(q * L, L)] = jnp.zeros((L,), jnp.float32)
            return c2
        lax.fori_loop(0, OUT // L, zlane, 0)
        return carry
    lax.fori_loop(0, _ZCH, zrow, 0)

    my_rows = N // NS
    zbase = c * N + s * my_rows
    for r0 in range(0, my_rows, _ZCH):
        pltpu.sync_copy(rows_v, agg_hbm.at[pl.ds(zbase + r0, _ZCH)])

    plsc.subcore_barrier()

    wid = s * NC + c

    def chunk(j, carry):
        base = wid * _SEG_PER_T + j * _SEG_CH
        pltpu.sync_copy(col_hbm.at[pl.ds(base, _SEG_CH)], col_v)
        pltpu.sync_copy(row_hbm.at[pl.ds(base, _SEG_CH)], row_v)
        pltpu.sync_copy(ew_hbm.at[pl.ds(base, _SEG_CH)], ew_v)

        pltpu.async_copy(h_hbm.at[col_v], rows_v, sem).wait()

        def mk_idx(i, c2):
            sl = pl.ds(i * L, L)
            sidx_v[sl] = row_v[sl] + c * N
            return c2
        lax.fori_loop(0, _SEG_CH // L, mk_idx, 0)

        def scale(g, c2):
            wv = ew_v[pl.ds(g * L, L)]
            for i in range(L):
                w = wv[i]
                e = g * L + i
                for q in range(OUT // L):
                    sl = pl.ds(q * L, L)
                    rows_v[e, sl] = rows_v[e, sl] * w
            return c2
        lax.fori_loop(0, _SEG_CH // L, scale, 0)

        pltpu.async_copy(rows_v, agg_hbm.at[sidx_v], sem, add=True).wait()
        return carry

    lax.fori_loop(0, _SEG_NCH, chunk, 0)


def _segment_sum(h, row, col, edge_w):
    k = functools.partial(
        pl.kernel,
        out_type=jax.ShapeDtypeStruct((2 * N, OUT), jnp.float32),
        mesh=plsc.VectorSubcoreMesh(core_axis_name="c", subcore_axis_name="s"),
        scratch_types=[
            pltpu.VMEM((_SEG_CH,), jnp.int32),
            pltpu.VMEM((_SEG_CH,), jnp.int32),
            pltpu.VMEM((_SEG_CH,), jnp.float32),
            pltpu.VMEM((_SEG_CH,), jnp.int32),
            pltpu.VMEM((_SEG_CH, OUT), jnp.float32),
            pltpu.SemaphoreType.DMA,
        ],
    )(_seg_kernel)
    return k(h, row, col, edge_w)


_G_CH = 128
_G_PER_W = B // (NC * NS)
_G_NCH = _G_PER_W // _G_CH


def _lookup_kernel(normed_hbm, x_hbm, out_hbm, x_v, idx_v, rows_v, sem):
    wid = lax.axis_index("s") * NC + lax.axis_index("c")
    base0 = wid * _G_PER_W

    def chunk(j, carry):
        base = base0 + j * _G_CH
        pltpu.sync_copy(x_hbm.at[pl.ds(base, _G_CH)], x_v)

        def mk_idx(i, c2):
            sl = pl.ds(i * L, L)
            t = x_v[sl] - 1
            idx_v[sl] = jnp.where(t < 0, t + N, t)
            return c2
        lax.fori_loop(0, _G_CH // L, mk_idx, 0)

        pltpu.async_copy(normed_hbm.at[idx_v], rows_v, sem).wait()
        pltpu.sync_copy(rows_v, out_hbm.at[pl.ds(base, _G_CH)])
        return carry

    lax.fori_loop(0, _G_NCH, chunk, 0)


def _lookup(normed, x):
    k = functools.partial(
        pl.kernel,
        out_type=jax.ShapeDtypeStruct((B, OUT), jnp.float32),
        mesh=plsc.VectorSubcoreMesh(core_axis_name="c", subcore_axis_name="s"),
        scratch_types=[
            pltpu.VMEM((_G_CH,), jnp.int32),
            pltpu.VMEM((_G_CH,), jnp.int32),
            pltpu.VMEM((_G_CH, OUT), jnp.float32),
            pltpu.SemaphoreType.DMA,
        ],
    )(_lookup_kernel)
    return k(normed, x)


def kernel(x, embed, attn, weight, ln_scale, ln_bias, G_values, G_indices):
    row = G_indices[0]
    col = G_indices[1]
    attn_flat = attn.reshape(N * N)

    h, l2s = _mm_l2(embed, attn, weight)
    edge_w = _edge_weights(row, col, G_values, attn_flat)
    agg2 = _segment_sum(h, row, col, edge_w)
    normed = _layernorm(agg2, ln_scale, ln_bias)
    res = _lookup(normed, x)
    return (res, l2s[0, 0])
